# baseline (speedup 1.0000x reference)
import jax
import jax.numpy as jnp
from jax.experimental import pallas as pl
from jax.experimental.pallas import tpu as pltpu

_B, _T, _N = 2, 2048, 64
_DS, _DR, _H = 1024, 256, 128
_K = 32
_TBLK = 64


def _bf16_dot(a, b):
    return jnp.dot(a.astype(jnp.bfloat16), b.astype(jnp.bfloat16),
                   preferred_element_type=jnp.float32)


def _body(s_ref, r_ref, wctx_ref, wslot_ref, w1_ref, b1_ref, w2_ref,
          out_ref, carry_ref):
    i = pl.program_id(1)

    @pl.when(i == 0)
    def _():
        carry_ref[...] = jnp.zeros_like(carry_ref)

    s = s_ref[0]
    row = jax.lax.broadcasted_iota(jnp.int32, (_TBLK, _TBLK), 0)
    col = jax.lax.broadcasted_iota(jnp.int32, (_TBLK, _TBLK), 1)
    tri = (row >= col).astype(jnp.float32)
    csum = jnp.dot(tri, s, preferred_element_type=jnp.float32,
                   precision=jax.lax.Precision.HIGHEST) + carry_ref[...]
    carry_ref[...] = csum[_TBLK - 1:_TBLK, :]
    tpos = jax.lax.broadcasted_iota(jnp.int32, (_TBLK, 1), 0)
    denom = (i * _TBLK + tpos + 1).astype(jnp.float32)
    ctx = _bf16_dot(csum / denom, wctx_ref[...])

    r2 = r_ref[0]
    rw = _bf16_dot(r2, wslot_ref[...])
    sf = rw.reshape(_TBLK, _N, _DR) + ctx[:, None, :]
    h = jnp.tanh(sf).reshape(_TBLK * _N, _DR)
    h = jnp.maximum(_bf16_dot(h, w1_ref[...]) + b1_ref[...], 0.0)
    hb = h.astype(jnp.bfloat16).astype(jnp.float32).reshape(_TBLK, _N, _H)
    w2b = w2_ref[...].astype(jnp.bfloat16).astype(jnp.float32)
    sc = jnp.sum(hb * w2b.reshape(1, 1, _H), axis=2)

    si = sc[:, :, None]
    sj = sc[:, None, :]
    jidx = jax.lax.broadcasted_iota(jnp.int32, (1, _N, _N), 2)
    iidx = jax.lax.broadcasted_iota(jnp.int32, (1, _N, _N), 1)
    beats = (sj > si) | ((sj == si) & (jidx < iidx))
    rank = jnp.sum(beats.astype(jnp.float32), axis=2)
    out_ref[0] = (rank < _K).astype(jnp.float32)


def kernel(s, r, W_ctx, W_slot, W1, b1, W2, b2):
    del b2
    grid = (_B, _T // _TBLK)
    out = pl.pallas_call(
        _body,
        grid=grid,
        in_specs=[
            pl.BlockSpec((1, _TBLK, _DS), lambda b, i: (b, i, 0)),
            pl.BlockSpec((1, _TBLK * _N, _DR), lambda b, i: (b, i, 0)),
            pl.BlockSpec((_DS, _DR), lambda b, i: (0, 0)),
            pl.BlockSpec((_DR, _DR), lambda b, i: (0, 0)),
            pl.BlockSpec((_DR, _H), lambda b, i: (0, 0)),
            pl.BlockSpec((1, _H), lambda b, i: (0, 0)),
            pl.BlockSpec((1, _H), lambda b, i: (0, 0)),
        ],
        out_specs=pl.BlockSpec((1, _TBLK, _N), lambda b, i: (b, i, 0)),
        out_shape=jax.ShapeDtypeStruct((_B, _T, _N), jnp.float32),
        scratch_shapes=[pltpu.VMEM((1, _DS), jnp.float32)],
    )(s, r.reshape(_B, _T * _N, _DR), W_ctx, W_slot, W1,
      b1.reshape(1, _H), W2.reshape(1, _H))
    return out[..., None]


# speedup vs baseline: 3.1870x; 3.1870x over previous
import jax
import jax.numpy as jnp
from jax.experimental import pallas as pl
from jax.experimental.pallas import tpu as pltpu

_B, _T, _N = 2, 2048, 64
_DS, _DR, _H = 1024, 256, 128
_K = 32
_TBLK = 64


def _bf16_dot(a, b):
    return jnp.dot(a, b, preferred_element_type=jnp.float32)


def _body(s_ref, r_ref, wctx_ref, wslot_ref, w1_ref, b1_ref, w2_ref,
          out_ref, carry_ref):
    i = pl.program_id(1)

    @pl.when(i == 0)
    def _():
        carry_ref[...] = jnp.zeros_like(carry_ref)

    s = s_ref[0]
    row = jax.lax.broadcasted_iota(jnp.int32, (_TBLK, _TBLK), 0)
    col = jax.lax.broadcasted_iota(jnp.int32, (_TBLK, _TBLK), 1)
    tri = (row >= col).astype(jnp.float32)
    csum = jnp.dot(tri, s, preferred_element_type=jnp.float32,
                   precision=jax.lax.Precision.HIGHEST) + carry_ref[...]
    carry_ref[...] = csum[_TBLK - 1:_TBLK, :]
    tpos = jax.lax.broadcasted_iota(jnp.int32, (_TBLK, 1), 0)
    denom = (i * _TBLK + tpos + 1).astype(jnp.float32)
    ctx = _bf16_dot(csum / denom, wctx_ref[...])

    r2 = r_ref[0]
    rw = _bf16_dot(r2, wslot_ref[...])
    sf = rw.reshape(_TBLK, _N, _DR) + ctx[:, None, :]
    h = jnp.tanh(sf).reshape(_TBLK * _N, _DR)
    h = jnp.maximum(_bf16_dot(h, w1_ref[...]) + b1_ref[...], 0.0)
    hb = h.astype(jnp.bfloat16).astype(jnp.float32).reshape(_TBLK, _N, _H)
    w2b = w2_ref[...].astype(jnp.bfloat16).astype(jnp.float32)
    sc = jnp.sum(hb * w2b.reshape(1, 1, _H), axis=2)

    lane = jax.lax.broadcasted_iota(jnp.int32, (_TBLK, _N), 1)
    rank = jnp.zeros((_TBLK, _N), jnp.float32)
    for d in range(1, _N):
        rd = jnp.roll(sc, -d, axis=1)
        beats = (rd > sc) | ((rd == sc) & (lane >= _N - d))
        rank = rank + beats.astype(jnp.float32)
    out_ref[0] = (rank < _K).astype(jnp.float32)


def kernel(s, r, W_ctx, W_slot, W1, b1, W2, b2):
    del b2
    grid = (_B, _T // _TBLK)
    out = pl.pallas_call(
        _body,
        grid=grid,
        in_specs=[
            pl.BlockSpec((1, _TBLK, _DS), lambda b, i: (b, i, 0)),
            pl.BlockSpec((1, _TBLK * _N, _DR), lambda b, i: (b, i, 0)),
            pl.BlockSpec((_DS, _DR), lambda b, i: (0, 0)),
            pl.BlockSpec((_DR, _DR), lambda b, i: (0, 0)),
            pl.BlockSpec((_DR, _H), lambda b, i: (0, 0)),
            pl.BlockSpec((1, _H), lambda b, i: (0, 0)),
            pl.BlockSpec((1, _H), lambda b, i: (0, 0)),
        ],
        out_specs=pl.BlockSpec((1, _TBLK, _N), lambda b, i: (b, i, 0)),
        out_shape=jax.ShapeDtypeStruct((_B, _T, _N), jnp.float32),
        scratch_shapes=[pltpu.VMEM((1, _DS), jnp.float32)],
    )(s, r.reshape(_B, _T * _N, _DR), W_ctx, W_slot, W1,
      b1.reshape(1, _H), W2.reshape(1, _H))
    return out[..., None]


# speedup vs baseline: 3.1872x; 1.0001x over previous
import jax
import jax.numpy as jnp
from jax.experimental import pallas as pl
from jax.experimental.pallas import tpu as pltpu

_B, _T, _N = 2, 2048, 64
_DS, _DR, _H = 1024, 256, 128
_K = 32
_TBLK = 64


def _bf16_dot(a, b):
    return jnp.dot(a, b, preferred_element_type=jnp.float32)


def _body(s_ref, r_ref, wctx_ref, wslot_ref, w1_ref, b1_ref, w2_ref,
          out_ref, carry_ref):
    i = pl.program_id(1)

    @pl.when(i == 0)
    def _():
        carry_ref[...] = jnp.zeros_like(carry_ref)

    s = s_ref[0]
    row = jax.lax.broadcasted_iota(jnp.int32, (_TBLK, _TBLK), 0)
    col = jax.lax.broadcasted_iota(jnp.int32, (_TBLK, _TBLK), 1)
    tri = (row >= col).astype(jnp.float32)
    csum = jnp.dot(tri, s, preferred_element_type=jnp.float32,
                   precision=jax.lax.Precision.HIGHEST) + carry_ref[...]
    carry_ref[...] = csum[_TBLK - 1:_TBLK, :]
    tpos = jax.lax.broadcasted_iota(jnp.int32, (_TBLK, 1), 0)
    denom = (i * _TBLK + tpos + 1).astype(jnp.float32)
    ctx = _bf16_dot(csum / denom, wctx_ref[...])

    r2 = r_ref[0]
    rw = _bf16_dot(r2, wslot_ref[...])
    sf = rw.reshape(_TBLK, _N, _DR) + ctx[:, None, :]
    h = jnp.tanh(sf).reshape(_TBLK * _N, _DR)
    h = jnp.maximum(_bf16_dot(h, w1_ref[...]) + b1_ref[...], 0.0)
    hb = h.astype(jnp.bfloat16).astype(jnp.float32).reshape(_TBLK, _N, _H)
    w2b = w2_ref[...].astype(jnp.bfloat16).astype(jnp.float32)
    sc = jnp.sum(hb * w2b.reshape(1, 1, _H), axis=2)

    lane = jax.lax.broadcasted_iota(jnp.int32, (_TBLK, _N), 1)
    rank = jnp.zeros((_TBLK, _N), jnp.float32)
    for d in range(1, _N):
        rd = jnp.roll(sc, -d, axis=1)
        beats = (rd > sc) | ((rd == sc) & (lane >= _N - d))
        rank = rank + beats.astype(jnp.float32)
    out_ref[0] = (rank < _K).astype(jnp.float32)


def kernel(s, r, W_ctx, W_slot, W1, b1, W2, b2):
    del b2
    grid = (_B, _T // _TBLK)
    out = pl.pallas_call(
        _body,
        grid=grid,
        in_specs=[
            pl.BlockSpec((1, _TBLK, _DS), lambda b, i: (b, i, 0)),
            pl.BlockSpec((1, _TBLK * _N, _DR), lambda b, i: (b, i, 0)),
            pl.BlockSpec((_DS, _DR), lambda b, i: (0, 0)),
            pl.BlockSpec((_DR, _DR), lambda b, i: (0, 0)),
            pl.BlockSpec((_DR, _H), lambda b, i: (0, 0)),
            pl.BlockSpec((1, _H), lambda b, i: (0, 0)),
            pl.BlockSpec((1, _H), lambda b, i: (0, 0)),
        ],
        out_specs=pl.BlockSpec((1, _TBLK, _N), lambda b, i: (b, i, 0)),
        out_shape=jax.ShapeDtypeStruct((_B, _T, _N), jnp.float32),
        scratch_shapes=[pltpu.VMEM((1, _DS), jnp.float32)],
        compiler_params=pltpu.CompilerParams(
            dimension_semantics=("parallel", "arbitrary")),
    )(s, r.reshape(_B, _T * _N, _DR), W_ctx, W_slot, W1,
      b1.reshape(1, _H), W2.reshape(1, _H))
    return out[..., None]


# speedup vs baseline: 20.2842x; 6.3643x over previous
import jax
import jax.numpy as jnp
from jax.experimental import pallas as pl
from jax.experimental.pallas import tpu as pltpu

_B, _T, _N = 2, 2048, 64
_DS, _DR, _H = 1024, 256, 128
_K = 32
_TBLK = 64


def _bf16_dot(a, b):
    return jnp.dot(a, b, preferred_element_type=jnp.float32)


def _body(s_ref, r_ref, wctx_ref, wslot_ref, w1_ref, b1_ref, w2_ref,
          out_ref, carry_ref):
    i = pl.program_id(1)

    @pl.when(i == 0)
    def _():
        carry_ref[...] = jnp.zeros_like(carry_ref)

    s = s_ref[0]
    row = jax.lax.broadcasted_iota(jnp.int32, (_TBLK, _TBLK), 0)
    col = jax.lax.broadcasted_iota(jnp.int32, (_TBLK, _TBLK), 1)
    tri = (row >= col).astype(jnp.float32)
    csum = jnp.dot(tri, s, preferred_element_type=jnp.float32,
                   precision=jax.lax.Precision.HIGHEST) + carry_ref[...]
    carry_ref[...] = csum[_TBLK - 1:_TBLK, :]
    tpos = jax.lax.broadcasted_iota(jnp.int32, (_TBLK, 1), 0)
    denom = (i * _TBLK + tpos + 1).astype(jnp.float32)
    ctx = _bf16_dot(csum / denom, wctx_ref[...])

    r2 = r_ref[0]
    rw = _bf16_dot(r2, wslot_ref[...])
    sf = rw.reshape(_TBLK, _N, _DR) + ctx[:, None, :]
    h = jnp.tanh(sf).reshape(_TBLK * _N, _DR)
    h = jnp.maximum(_bf16_dot(h, w1_ref[...]) + b1_ref[...], 0.0)
    q = _bf16_dot(h, jnp.broadcast_to(w2_ref[...], (_H, _N)))
    q3 = q.reshape(_TBLK, _N, _N)
    n1 = jax.lax.broadcasted_iota(jnp.int32, (1, _N, _N), 1)
    m1 = jax.lax.broadcasted_iota(jnp.int32, (1, _N, _N), 2)
    eye = (n1 == m1).astype(jnp.float32)
    sc = jnp.sum(q3 * eye, axis=1)

    lane = jax.lax.broadcasted_iota(jnp.int32, (_TBLK, _N), 1)
    rank = jnp.zeros((_TBLK, _N), jnp.float32)
    for d in range(1, _N):
        rd = jnp.roll(sc, -d, axis=1)
        beats = (rd > sc) | ((rd == sc) & (lane >= _N - d))
        rank = rank + beats.astype(jnp.float32)
    out_ref[0] = (rank < _K).astype(jnp.float32)


def kernel(s, r, W_ctx, W_slot, W1, b1, W2, b2):
    del b2
    grid = (_B, _T // _TBLK)
    out = pl.pallas_call(
        _body,
        grid=grid,
        in_specs=[
            pl.BlockSpec((1, _TBLK, _DS), lambda b, i: (b, i, 0)),
            pl.BlockSpec((1, _TBLK * _N, _DR), lambda b, i: (b, i, 0)),
            pl.BlockSpec((_DS, _DR), lambda b, i: (0, 0)),
            pl.BlockSpec((_DR, _DR), lambda b, i: (0, 0)),
            pl.BlockSpec((_DR, _H), lambda b, i: (0, 0)),
            pl.BlockSpec((1, _H), lambda b, i: (0, 0)),
            pl.BlockSpec((_H, 1), lambda b, i: (0, 0)),
        ],
        out_specs=pl.BlockSpec((1, _TBLK, _N), lambda b, i: (b, i, 0)),
        out_shape=jax.ShapeDtypeStruct((_B, _T, _N), jnp.float32),
        scratch_shapes=[pltpu.VMEM((1, _DS), jnp.float32)],
        compiler_params=pltpu.CompilerParams(
            dimension_semantics=("parallel", "arbitrary")),
    )(s, r.reshape(_B, _T * _N, _DR), W_ctx, W_slot, W1,
      b1.reshape(1, _H), W2)
    return out[..., None]


# speedup vs baseline: 21.3975x; 1.0549x over previous
import jax
import jax.numpy as jnp
from jax.experimental import pallas as pl
from jax.experimental.pallas import tpu as pltpu

_B, _T, _N = 2, 2048, 64
_DS, _DR, _H = 1024, 256, 128
_K = 32
_TBLK = 128


def _bf16_dot(a, b):
    return jnp.dot(a, b, preferred_element_type=jnp.float32)


def _body(s_ref, r_ref, wctx_ref, wslot_ref, w1_ref, b1_ref, w2_ref,
          out_ref, carry_ref):
    i = pl.program_id(1)

    @pl.when(i == 0)
    def _():
        carry_ref[...] = jnp.zeros_like(carry_ref)

    s = s_ref[0]
    row = jax.lax.broadcasted_iota(jnp.int32, (_TBLK, _TBLK), 0)
    col = jax.lax.broadcasted_iota(jnp.int32, (_TBLK, _TBLK), 1)
    tri = (row >= col).astype(jnp.float32)
    csum = jnp.dot(tri, s, preferred_element_type=jnp.float32,
                   precision=jax.lax.Precision.HIGHEST) + carry_ref[...]
    carry_ref[...] = csum[_TBLK - 1:_TBLK, :]
    tpos = jax.lax.broadcasted_iota(jnp.int32, (_TBLK, 1), 0)
    denom = (i * _TBLK + tpos + 1).astype(jnp.float32)
    ctx = _bf16_dot(csum / denom, wctx_ref[...])

    r2 = r_ref[0]
    rw = _bf16_dot(r2, wslot_ref[...])
    sf = rw.reshape(_TBLK, _N, _DR) + ctx[:, None, :]
    h = jnp.tanh(sf).reshape(_TBLK * _N, _DR)
    h = jnp.maximum(_bf16_dot(h, w1_ref[...]) + b1_ref[...], 0.0)
    q = _bf16_dot(h, jnp.broadcast_to(w2_ref[...], (_H, _N)))
    q3 = q.reshape(_TBLK, _N, _N)
    n1 = jax.lax.broadcasted_iota(jnp.int32, (1, _N, _N), 1)
    m1 = jax.lax.broadcasted_iota(jnp.int32, (1, _N, _N), 2)
    eye = (n1 == m1).astype(jnp.float32)
    sc = jnp.sum(q3 * eye, axis=1)

    lane = jax.lax.broadcasted_iota(jnp.int32, (_TBLK, _N), 1)
    rank = jnp.zeros((_TBLK, _N), jnp.float32)
    for d in range(1, _N):
        rd = jnp.roll(sc, -d, axis=1)
        beats = (rd > sc) | ((rd == sc) & (lane >= _N - d))
        rank = rank + beats.astype(jnp.float32)
    out_ref[0] = (rank < _K).astype(jnp.float32)


def kernel(s, r, W_ctx, W_slot, W1, b1, W2, b2):
    del b2
    grid = (_B, _T // _TBLK)
    out = pl.pallas_call(
        _body,
        grid=grid,
        in_specs=[
            pl.BlockSpec((1, _TBLK, _DS), lambda b, i: (b, i, 0)),
            pl.BlockSpec((1, _TBLK * _N, _DR), lambda b, i: (b, i, 0)),
            pl.BlockSpec((_DS, _DR), lambda b, i: (0, 0)),
            pl.BlockSpec((_DR, _DR), lambda b, i: (0, 0)),
            pl.BlockSpec((_DR, _H), lambda b, i: (0, 0)),
            pl.BlockSpec((1, _H), lambda b, i: (0, 0)),
            pl.BlockSpec((_H, 1), lambda b, i: (0, 0)),
        ],
        out_specs=pl.BlockSpec((1, _TBLK, _N), lambda b, i: (b, i, 0)),
        out_shape=jax.ShapeDtypeStruct((_B, _T, _N), jnp.float32),
        scratch_shapes=[pltpu.VMEM((1, _DS), jnp.float32)],
        compiler_params=pltpu.CompilerParams(
            dimension_semantics=("parallel", "arbitrary")),
    )(s, r.reshape(_B, _T * _N, _DR), W_ctx, W_slot, W1,
      b1.reshape(1, _H), W2)
    return out[..., None]
